# baseline (speedup 1.0000x reference)
import dataclasses
import functools

import jax
import jax.numpy as jnp
from jax import lax
from jax.experimental import pallas as pl
from jax.experimental.pallas import tpu as pltpu
from jax.experimental.pallas import tpu_sc as plsc

N_NODES = 65536
B = 16
P = 2048
D = 128
IN_FEAT = 2 * D

NC = 2
NS = 16
L = 16
NW = NC * NS
ROWS = B * P
ROWS_PER_TILE = ROWS // NW
CHUNK = 128
NCHUNK = ROWS_PER_TILE // CHUNK


def _gather_sc(graph1_x, graph2_x, idx_l, idx_r, g1_len, g2_len):
    mesh = plsc.VectorSubcoreMesh(core_axis_name="c", subcore_axis_name="s")
    cp = pltpu.CompilerParams()
    if "needs_layout_passes" in pltpu.CompilerParams.__dataclass_fields__:
        cp = dataclasses.replace(cp, needs_layout_passes=False)

    @functools.partial(
        pl.kernel,
        out_type=jax.ShapeDtypeStruct((ROWS, IN_FEAT), jnp.bfloat16),
        mesh=mesh,
        compiler_params=cp,
        scratch_types=[
            pltpu.VMEM((L,), jnp.int32),
            pltpu.VMEM((L,), jnp.int32),
            pltpu.VMEM((ROWS_PER_TILE,), jnp.int32),
            pltpu.VMEM((ROWS_PER_TILE,), jnp.int32),
        ] + [pltpu.VMEM((CHUNK, D), jnp.float32) for _ in range(4)]
          + [pltpu.VMEM((CHUNK, D), jnp.bfloat16) for _ in range(4)]
          + [pltpu.SemaphoreType.DMA for _ in range(8)],
    )
    def k(t1_hbm, t2_hbm, il_hbm, ir_hbm, l1_hbm, l2_hbm,
          o_hbm,
          len1_v, len2_v, il_v, ir_v, *bufs_and_sems):
        bufs = bufs_and_sems[:4]
        bbufs = bufs_and_sems[4:8]
        gsems = bufs_and_sems[8:12]
        wsems = bufs_and_sems[12:16]
        wid = lax.axis_index("s") * NC + lax.axis_index("c")
        base = wid * ROWS_PER_TILE
        bidx = wid // (P // ROWS_PER_TILE)

        pltpu.sync_copy(l1_hbm, len1_v)
        pltpu.sync_copy(l2_hbm, len2_v)
        pltpu.sync_copy(il_hbm.at[pl.ds(base, ROWS_PER_TILE)], il_v)
        pltpu.sync_copy(ir_hbm.at[pl.ds(base, ROWS_PER_TILE)], ir_v)

        mask = lax.iota(jnp.int32, L) < bidx
        zeros = jnp.zeros((L,), jnp.int32)
        off1 = jnp.sum(jnp.where(mask, len1_v[...], zeros))
        off2 = jnp.sum(jnp.where(mask, len2_v[...], zeros))

        @pl.loop(0, ROWS_PER_TILE, step=L)
        def _(j):
            il_v[pl.ds(j, L)] = il_v[pl.ds(j, L)] + off1
            ir_v[pl.ds(j, L)] = ir_v[pl.ds(j, L)] + off2

        jobs = []
        for c in range(NCHUNK):
            jobs.append((t1_hbm, il_v, 0, c))
            jobs.append((t2_hbm, ir_v, D, c))
        NJOBS = len(jobs)
        NBUF, K = 4, 3

        def gstart(j):
            tbl, iv, _, c = jobs[j]
            i = j % NBUF
            return pltpu.async_copy(
                tbl.at[iv.at[pl.ds(c * CHUNK, CHUNK)]], bufs[i], gsems[i])

        def convert(i):
            fb = bufs[i]
            bb = bbufs[i]

            @pl.loop(0, CHUNK)
            def _(r):
                for c in range(0, D, 2 * L):
                    a = fb[r, pl.ds(c, L)]
                    b = fb[r, pl.ds(c + L, L)]
                    packed = plsc.pack(a, b, format=plsc.PackFormat.INTERLEAVED)
                    bb[r, pl.ds(c, 2 * L)] = packed

        gd = [None] * NJOBS
        wd = [None] * NJOBS
        for j in range(K):
            gd[j] = gstart(j)
        for j in range(NJOBS):
            i = j % NBUF
            gd[j].wait()
            if j >= NBUF:
                wd[j - NBUF].wait()
                wd[j - NBUF] = None
            convert(i)
            _, _, col, c = jobs[j]
            wd[j] = pltpu.async_copy(
                bbufs[i],
                o_hbm.at[pl.ds(base + c * CHUNK, CHUNK), pl.ds(col, D)],
                wsems[i])
            jn = j + K
            if jn < NJOBS:
                gd[jn] = gstart(jn)
        for j in range(NJOBS):
            if wd[j] is not None:
                wd[j].wait()

    return k(graph1_x, graph2_x, idx_l, idx_r, g1_len, g2_len)


def _mlp_tc(x, w1, b1, w2, b2, BLK=4096):

    def body(x_ref, w1_ref, b1_ref, w2_ref, b2_ref, o_ref):
        xb = x_ref[...]
        w1b_ = w1_ref[...].astype(jnp.bfloat16)
        h = jnp.dot(xb, w1b_, preferred_element_type=jnp.float32) + b1_ref[...]
        h = jnp.maximum(h, 0.0)
        w2row = jnp.transpose(w2_ref[...])
        o_ref[...] = jnp.sum(h * w2row, axis=1, keepdims=True) + b2_ref[0, 0]

    return pl.pallas_call(
        body,
        grid=(ROWS // BLK,),
        in_specs=[
            pl.BlockSpec((BLK, IN_FEAT), lambda i: (i, 0)),
            pl.BlockSpec((IN_FEAT, IN_FEAT), lambda i: (0, 0)),
            pl.BlockSpec((1, IN_FEAT), lambda i: (0, 0)),
            pl.BlockSpec((IN_FEAT, 1), lambda i: (0, 0)),
            pl.BlockSpec((1, 1), lambda i: (0, 0)),
        ],
        out_specs=pl.BlockSpec((BLK, 1), lambda i: (i, 0)),
        out_shape=jax.ShapeDtypeStruct((ROWS, 1), jnp.float32),
        compiler_params=pltpu.CompilerParams(
            dimension_semantics=("parallel",)),
    )(x, w1, b1, w2, b2)


_PERM = []
for _g in range(IN_FEAT // (2 * L)):
    for _i in range(L):
        _PERM.extend([_g * 2 * L + _i, _g * 2 * L + L + _i])
_PERM = tuple(_PERM)


def kernel(graph1_x, graph2_x, idx_left, idx_right, g1_len, g2_len, W1, b1, W2, b2):
    il = idx_left.reshape(-1)
    ir = idx_right.reshape(-1)
    x = _gather_sc(graph1_x, graph2_x, il, ir, g1_len, g2_len)
    w1p = W1[jnp.array(_PERM, dtype=jnp.int32)]
    out = _mlp_tc(x, w1p, b1.reshape(1, IN_FEAT), W2, b2.reshape(1, 1), BLK=8192)
    return out


# speedup vs baseline: 1.1952x; 1.1952x over previous
import dataclasses
import functools

import jax
import jax.numpy as jnp
from jax import lax
from jax.experimental import pallas as pl
from jax.experimental.pallas import tpu as pltpu
from jax.experimental.pallas import tpu_sc as plsc

N_NODES = 65536
B = 16
P = 2048
D = 128
IN_FEAT = 2 * D

NC = 2
NS = 16
L = 16
NW = NC * NS
ROWS = B * P
ROWS_PER_TILE = ROWS // NW
CHUNK = 128
NCHUNK = ROWS_PER_TILE // CHUNK


def _gather_sc(graph1_x, graph2_x, idx_l, idx_r, g1_len, g2_len):
    mesh = plsc.VectorSubcoreMesh(core_axis_name="c", subcore_axis_name="s")
    cp = pltpu.CompilerParams()
    if "needs_layout_passes" in pltpu.CompilerParams.__dataclass_fields__:
        cp = dataclasses.replace(cp, needs_layout_passes=False)

    @functools.partial(
        pl.kernel,
        out_type=jax.ShapeDtypeStruct((ROWS, IN_FEAT), jnp.float32),
        mesh=mesh,
        compiler_params=cp,
        scratch_types=[
            pltpu.VMEM((L,), jnp.int32),
            pltpu.VMEM((L,), jnp.int32),
            pltpu.VMEM((ROWS_PER_TILE,), jnp.int32),
            pltpu.VMEM((ROWS_PER_TILE,), jnp.int32),
        ] + [pltpu.VMEM((CHUNK, D), jnp.float32) for _ in range(7)]
          + [pltpu.SemaphoreType.DMA for _ in range(14)],
    )
    def k(t1_hbm, t2_hbm, il_hbm, ir_hbm, l1_hbm, l2_hbm,
          o_hbm,
          len1_v, len2_v, il_v, ir_v, *bufs_and_sems):
        bufs = bufs_and_sems[:7]
        gsems = bufs_and_sems[7:14]
        wsems = bufs_and_sems[14:21]
        wid = lax.axis_index("s") * NC + lax.axis_index("c")
        base = wid * ROWS_PER_TILE
        bidx = wid // (P // ROWS_PER_TILE)

        pltpu.sync_copy(l1_hbm, len1_v)
        pltpu.sync_copy(l2_hbm, len2_v)
        pltpu.sync_copy(il_hbm.at[pl.ds(base, ROWS_PER_TILE)], il_v)
        pltpu.sync_copy(ir_hbm.at[pl.ds(base, ROWS_PER_TILE)], ir_v)

        mask = lax.iota(jnp.int32, L) < bidx
        zeros = jnp.zeros((L,), jnp.int32)
        off1 = jnp.sum(jnp.where(mask, len1_v[...], zeros))
        off2 = jnp.sum(jnp.where(mask, len2_v[...], zeros))

        @pl.loop(0, ROWS_PER_TILE, step=L)
        def _(j):
            il_v[pl.ds(j, L)] = il_v[pl.ds(j, L)] + off1
            ir_v[pl.ds(j, L)] = ir_v[pl.ds(j, L)] + off2

        jobs = []
        for c in range(NCHUNK):
            jobs.append((t1_hbm, il_v, 0, c))
            jobs.append((t2_hbm, ir_v, D, c))
        NJOBS = len(jobs)
        NBUF, K = 7, 4

        def gstart(j):
            tbl, iv, _, c = jobs[j]
            i = j % NBUF
            return pltpu.async_copy(
                tbl.at[iv.at[pl.ds(c * CHUNK, CHUNK)]], bufs[i], gsems[i])

        gd = [None] * NJOBS
        wd = [None] * NJOBS
        for j in range(K):
            gd[j] = gstart(j)
        for j in range(NJOBS):
            i = j % NBUF
            gd[j].wait()
            _, _, col, c = jobs[j]
            wd[j] = pltpu.async_copy(
                bufs[i],
                o_hbm.at[pl.ds(base + c * CHUNK, CHUNK), pl.ds(col, D)],
                wsems[i])
            jn = j + K
            if jn < NJOBS:
                if jn >= NBUF:
                    wd[jn - NBUF].wait()
                    wd[jn - NBUF] = None
                gd[jn] = gstart(jn)
        for j in range(NJOBS):
            if wd[j] is not None:
                wd[j].wait()

    return k(graph1_x, graph2_x, idx_l, idx_r, g1_len, g2_len)


def _mlp_tc(x, w1, b1, w2, b2, BLK=4096):

    def body(x_ref, w1_ref, b1_ref, w2_ref, b2_ref, o_ref):
        xb = x_ref[...].astype(jnp.bfloat16)
        w1b_ = w1_ref[...].astype(jnp.bfloat16)
        h = jnp.dot(xb, w1b_, preferred_element_type=jnp.float32) + b1_ref[...]
        h = jnp.maximum(h, 0.0)
        w2row = jnp.transpose(w2_ref[...])
        o_ref[...] = jnp.sum(h * w2row, axis=1, keepdims=True) + b2_ref[0, 0]

    return pl.pallas_call(
        body,
        grid=(ROWS // BLK,),
        in_specs=[
            pl.BlockSpec((BLK, IN_FEAT), lambda i: (i, 0)),
            pl.BlockSpec((IN_FEAT, IN_FEAT), lambda i: (0, 0)),
            pl.BlockSpec((1, IN_FEAT), lambda i: (0, 0)),
            pl.BlockSpec((IN_FEAT, 1), lambda i: (0, 0)),
            pl.BlockSpec((1, 1), lambda i: (0, 0)),
        ],
        out_specs=pl.BlockSpec((BLK, 1), lambda i: (i, 0)),
        out_shape=jax.ShapeDtypeStruct((ROWS, 1), jnp.float32),
        compiler_params=pltpu.CompilerParams(
            dimension_semantics=("parallel",)),
    )(x, w1, b1, w2, b2)


def kernel(graph1_x, graph2_x, idx_left, idx_right, g1_len, g2_len, W1, b1, W2, b2):
    il = idx_left.reshape(-1)
    ir = idx_right.reshape(-1)
    x = _gather_sc(graph1_x, graph2_x, il, ir, g1_len, g2_len)
    out = _mlp_tc(x, W1, b1.reshape(1, IN_FEAT), W2, b2.reshape(1, 1), BLK=8192)
    return out


# speedup vs baseline: 1.2151x; 1.0166x over previous
import dataclasses
import functools

import jax
import jax.numpy as jnp
from jax import lax
from jax.experimental import pallas as pl
from jax.experimental.pallas import tpu as pltpu
from jax.experimental.pallas import tpu_sc as plsc

N_NODES = 65536
B = 16
P = 2048
D = 128
IN_FEAT = 2 * D

NC = 2
NS = 16
L = 16
NW = NC * NS
ROWS = B * P
ROWS_PER_TILE = ROWS // NW
CHUNK = 128
NCHUNK = ROWS_PER_TILE // CHUNK


def _gather_sc(graph1_x, graph2_x, idx_l, idx_r, g1_len, g2_len):
    mesh = plsc.VectorSubcoreMesh(core_axis_name="c", subcore_axis_name="s")
    cp = pltpu.CompilerParams()
    if "needs_layout_passes" in pltpu.CompilerParams.__dataclass_fields__:
        cp = dataclasses.replace(cp, needs_layout_passes=False)

    @functools.partial(
        pl.kernel,
        out_type=jax.ShapeDtypeStruct((ROWS, IN_FEAT), jnp.float32),
        mesh=mesh,
        compiler_params=cp,
        scratch_types=[
            pltpu.VMEM((L,), jnp.int32),
            pltpu.VMEM((L,), jnp.int32),
            pltpu.VMEM((ROWS_PER_TILE,), jnp.int32),
            pltpu.VMEM((ROWS_PER_TILE,), jnp.int32),
        ] + [pltpu.VMEM((CHUNK, D), jnp.float32) for _ in range(7)]
          + [pltpu.SemaphoreType.DMA for _ in range(14)],
    )
    def k(t1_hbm, t2_hbm, il_hbm, ir_hbm, l1_hbm, l2_hbm,
          o_hbm,
          len1_v, len2_v, il_v, ir_v, *bufs_and_sems):
        bufs = bufs_and_sems[:7]
        gsems = bufs_and_sems[7:14]
        wsems = bufs_and_sems[14:21]
        wid = lax.axis_index("s") * NC + lax.axis_index("c")
        base = wid * ROWS_PER_TILE
        bidx = wid // (P // ROWS_PER_TILE)

        pltpu.sync_copy(l1_hbm, len1_v)
        pltpu.sync_copy(l2_hbm, len2_v)
        pltpu.sync_copy(il_hbm.at[pl.ds(base, ROWS_PER_TILE)], il_v)
        pltpu.sync_copy(ir_hbm.at[pl.ds(base, ROWS_PER_TILE)], ir_v)

        mask = lax.iota(jnp.int32, L) < bidx
        zeros = jnp.zeros((L,), jnp.int32)
        off1 = jnp.sum(jnp.where(mask, len1_v[...], zeros))
        off2 = jnp.sum(jnp.where(mask, len2_v[...], zeros))

        @pl.loop(0, ROWS_PER_TILE, step=L)
        def _(j):
            il_v[pl.ds(j, L)] = il_v[pl.ds(j, L)] + off1
            ir_v[pl.ds(j, L)] = ir_v[pl.ds(j, L)] + off2

        jobs = []
        for c in range(NCHUNK):
            jobs.append((t1_hbm, il_v, 0, c))
            jobs.append((t2_hbm, ir_v, D, c))
        NJOBS = len(jobs)
        NBUF, K = 7, 5

        def gstart(j):
            tbl, iv, _, c = jobs[j]
            i = j % NBUF
            return pltpu.async_copy(
                tbl.at[iv.at[pl.ds(c * CHUNK, CHUNK)]], bufs[i], gsems[i])

        gd = [None] * NJOBS
        wd = [None] * NJOBS
        for j in range(K):
            gd[j] = gstart(j)
        for j in range(NJOBS):
            i = j % NBUF
            gd[j].wait()
            _, _, col, c = jobs[j]
            wd[j] = pltpu.async_copy(
                bufs[i],
                o_hbm.at[pl.ds(base + c * CHUNK, CHUNK), pl.ds(col, D)],
                wsems[i])
            jn = j + K
            if jn < NJOBS:
                if jn >= NBUF:
                    wd[jn - NBUF].wait()
                    wd[jn - NBUF] = None
                gd[jn] = gstart(jn)
        for j in range(NJOBS):
            if wd[j] is not None:
                wd[j].wait()

    return k(graph1_x, graph2_x, idx_l, idx_r, g1_len, g2_len)


def _mlp_tc(x, w1, b1, w2, b2, BLK=4096):

    def body(x_ref, w1_ref, b1_ref, w2_ref, b2_ref, o_ref):
        xb = x_ref[...].astype(jnp.bfloat16)
        w1b_ = w1_ref[...].astype(jnp.bfloat16)
        h = jnp.dot(xb, w1b_, preferred_element_type=jnp.float32) + b1_ref[...]
        h = jnp.maximum(h, 0.0)
        w2row = jnp.transpose(w2_ref[...])
        o_ref[...] = jnp.sum(h * w2row, axis=1, keepdims=True) + b2_ref[0, 0]

    return pl.pallas_call(
        body,
        grid=(ROWS // BLK,),
        in_specs=[
            pl.BlockSpec((BLK, IN_FEAT), lambda i: (i, 0)),
            pl.BlockSpec((IN_FEAT, IN_FEAT), lambda i: (0, 0)),
            pl.BlockSpec((1, IN_FEAT), lambda i: (0, 0)),
            pl.BlockSpec((IN_FEAT, 1), lambda i: (0, 0)),
            pl.BlockSpec((1, 1), lambda i: (0, 0)),
        ],
        out_specs=pl.BlockSpec((BLK, 1), lambda i: (i, 0)),
        out_shape=jax.ShapeDtypeStruct((ROWS, 1), jnp.float32),
        compiler_params=pltpu.CompilerParams(
            dimension_semantics=("parallel",)),
    )(x, w1, b1, w2, b2)


def kernel(graph1_x, graph2_x, idx_left, idx_right, g1_len, g2_len, W1, b1, W2, b2):
    il = idx_left.reshape(-1)
    ir = idx_right.reshape(-1)
    x = _gather_sc(graph1_x, graph2_x, il, ir, g1_len, g2_len)
    out = _mlp_tc(x, W1, b1.reshape(1, IN_FEAT), W2, b2.reshape(1, 1), BLK=8192)
    return out
